# Initial kernel scaffold; baseline (speedup 1.0000x reference)
import jax
import jax.numpy as jnp
from jax import lax
from jax.experimental import pallas as pl
from jax.experimental.pallas import tpu as pltpu
from jax.experimental.pallas import tpu_sc as plsc

N = 10000
E = 320000
B = 256
D = 128
ED = 16
EH = 32

NC = 2
NS = 16
NW = NC * NS
EPW = E // NW
ROWS_PER_TILE = N // NS

_HIGH = lax.Precision.HIGHEST


_EBLK = 32000


def _edge_mlp_body(ea_ref, mw1a, mb1a, mw2a, mb2a, mw1b, mb1b, mw2b, mb2b,
                   w1_ref, w2_ref):
    ea = ea_ref[...]

    def mlp(mw1, mb1, mw2, mb2):
        h = jnp.maximum(
            jnp.dot(ea, mw1[...], preferred_element_type=jnp.float32,
                    precision=_HIGH) + mb1[...], 0.0)
        z = jnp.dot(h, mw2[...], preferred_element_type=jnp.float32,
                    precision=_HIGH)[:, 0] + mb2[0]
        return jax.nn.sigmoid(z)

    w1_ref[...] = mlp(mw1a, mb1a, mw2a, mb2a)
    w2_ref[...] = mlp(mw1b, mb1b, mw2b, mb2b)


def _edge_weights(edge_attr, mw1a, mb1a, mw2a, mb2a, mw1b, mb1b, mw2b, mb2b):
    full = lambda *s: pl.BlockSpec(s, lambda i: tuple(0 for _ in s))
    return pl.pallas_call(
        _edge_mlp_body,
        grid=(E // _EBLK,),
        in_specs=[
            pl.BlockSpec((_EBLK, ED), lambda i: (i, 0)),
            full(ED, EH), full(EH), full(EH, 1), full(1),
            full(ED, EH), full(EH), full(EH, 1), full(1),
        ],
        out_specs=[
            pl.BlockSpec((_EBLK,), lambda i: (i,)),
            pl.BlockSpec((_EBLK,), lambda i: (i,)),
        ],
        out_shape=[
            jax.ShapeDtypeStruct((E,), jnp.float32),
            jax.ShapeDtypeStruct((E,), jnp.float32),
        ],
    )(edge_attr, mw1a, mb1a, mw2a, mb2a, mw1b, mb1b, mw2b, mb2b)


_DCHUNK = 2000


def _degree_partials(col, w1, w2):
    mesh = plsc.VectorSubcoreMesh(core_axis_name="c", subcore_axis_name="s")

    @pl.kernel(
        out_type=jax.ShapeDtypeStruct((NW, 2, N), jnp.float32),
        mesh=mesh,
        scratch_types=[
            pltpu.VMEM((_DCHUNK,), jnp.int32),
            pltpu.VMEM((_DCHUNK,), jnp.float32),
            pltpu.VMEM((_DCHUNK,), jnp.float32),
            pltpu.VMEM((N,), jnp.float32),
            pltpu.VMEM((N,), jnp.float32),
        ],
    )
    def deg_kernel(col_hbm, w1_hbm, w2_hbm, out_hbm,
                   colbuf, w1buf, w2buf, acc1, acc2):
        cid = lax.axis_index("c")
        sid = lax.axis_index("s")
        wid = sid * NC + cid
        base = wid * EPW
        zero = jnp.zeros((16,), jnp.float32)

        @pl.loop(0, N // 16)
        def _(g):
            acc1[pl.ds(g * 16, 16)] = zero
            acc2[pl.ds(g * 16, 16)] = zero

        @pl.loop(0, EPW // _DCHUNK)
        def _(i):
            off = base + i * _DCHUNK
            pltpu.sync_copy(col_hbm.at[pl.ds(off, _DCHUNK)], colbuf)
            pltpu.sync_copy(w1_hbm.at[pl.ds(off, _DCHUNK)], w1buf)
            pltpu.sync_copy(w2_hbm.at[pl.ds(off, _DCHUNK)], w2buf)

            @pl.loop(0, _DCHUNK // 16)
            def _(g):
                idx = colbuf[pl.ds(g * 16, 16)]
                plsc.addupdate_scatter(acc1, [idx], w1buf[pl.ds(g * 16, 16)])
                plsc.addupdate_scatter(acc2, [idx], w2buf[pl.ds(g * 16, 16)])

        pltpu.sync_copy(acc1, out_hbm.at[wid, 0])
        pltpu.sync_copy(acc2, out_hbm.at[wid, 1])

    return deg_kernel(col, w1, w2)


_K = 80


def _aggregate(xs, row, col, w):
    mesh = plsc.VectorSubcoreMesh(core_axis_name="c", subcore_axis_name="s")

    @pl.kernel(
        out_type=jax.ShapeDtypeStruct((NC, N, D), jnp.float32),
        mesh=mesh,
        scratch_types=[
            pltpu.VMEM((_K,), jnp.int32),
            pltpu.VMEM((_K,), jnp.int32),
            pltpu.VMEM((_K,), jnp.float32),
            pltpu.VMEM((_K, D), jnp.float32),
            pltpu.VMEM_SHARED((N, D), jnp.float32),
        ],
    )
    def agg_kernel(xs_hbm, row_hbm, col_hbm, w_hbm, out_hbm,
                   rowbuf, colbuf, wbuf, rows, acc):
        cid = lax.axis_index("c")
        sid = lax.axis_index("s")
        wid = sid * NC + cid
        base = wid * EPW
        zero = jnp.zeros((16,), jnp.float32)

        @pl.loop(0, _K)
        def _(r):
            for c2 in range(D // 16):
                rows[r, pl.ds(c2 * 16, 16)] = zero

        rbase = sid * ROWS_PER_TILE

        @pl.loop(0, 7)
        def _(i):
            pltpu.sync_copy(rows, acc.at[pl.ds(rbase + i * _K, _K)])

        pltpu.sync_copy(rows.at[pl.ds(0, 65)], acc.at[pl.ds(rbase + 560, 65)])
        plsc.subcore_barrier()

        @pl.loop(0, EPW // _K)
        def _(i):
            off = base + i * _K
            pltpu.sync_copy(row_hbm.at[pl.ds(off, _K)], rowbuf)
            pltpu.sync_copy(col_hbm.at[pl.ds(off, _K)], colbuf)
            pltpu.sync_copy(w_hbm.at[pl.ds(off, _K)], wbuf)
            pltpu.sync_copy(xs_hbm.at[rowbuf], rows)

            @pl.loop(0, _K)
            def _(j):
                jv = jnp.broadcast_to(j, (16,)).astype(jnp.int32)
                wv = plsc.load_gather(wbuf, [jv])
                for c2 in range(D // 16):
                    sl = (j, pl.ds(c2 * 16, 16))
                    rows[sl] = rows[sl] * wv

            pltpu.sync_copy(rows, acc.at[colbuf], add=True)

        plsc.subcore_barrier()
        pltpu.sync_copy(acc.at[pl.ds(rbase, ROWS_PER_TILE)],
                        out_hbm.at[cid, pl.ds(rbase, ROWS_PER_TILE)])

    return agg_kernel(xs, row, col, w)


def _prep1_body(x_ref, lw_ref, degp_ref, xs_ref, d1_ref, d2_ref):
    degs = jnp.sum(degp_ref[...], axis=0)
    d1 = lax.rsqrt(degs[0] + 1.0)
    d2 = lax.rsqrt(degs[1] + 1.0)
    d1_ref[...] = d1
    d2_ref[...] = d2
    xl = jnp.dot(x_ref[...], lw_ref[...], preferred_element_type=jnp.float32,
                 precision=_HIGH)
    xs_ref[...] = d1[:, None] * xl


def _prep1(x, lin_w, degp):
    return pl.pallas_call(
        _prep1_body,
        out_shape=[
            jax.ShapeDtypeStruct((N, D), jnp.float32),
            jax.ShapeDtypeStruct((N,), jnp.float32),
            jax.ShapeDtypeStruct((N,), jnp.float32),
        ],
    )(x, lin_w, degp)


def _mid_body(parts_ref, xs1_ref, d1_ref, d2_ref, c1b_ref, g1w_ref, g1b_ref,
              lw2_ref, xs2_ref):
    agg = parts_ref[0] + parts_ref[1] + xs1_ref[...]
    out1 = d1_ref[...][:, None] * agg + c1b_ref[...]
    h = jnp.maximum(
        jnp.dot(out1, g1w_ref[...], preferred_element_type=jnp.float32,
                precision=_HIGH) + g1b_ref[...], 0.0)
    xl2 = jnp.dot(h, lw2_ref[...], preferred_element_type=jnp.float32,
                  precision=_HIGH)
    xs2_ref[...] = d2_ref[...][:, None] * xl2


def _mid(parts1, xs1, d1, d2, c1_bias, g1_w, g1_b, c2_lin_w):
    return pl.pallas_call(
        _mid_body,
        out_shape=jax.ShapeDtypeStruct((N, D), jnp.float32),
    )(parts1, xs1, d1, d2, c1_bias, g1_w, g1_b, c2_lin_w)


def _head_body(parts_ref, xs2_ref, d2_ref, c2b_ref, g2w_ref, g2b_ref,
               bf_ref, mol_ref, m0w, m0b, m1w, m1b, m2w, m2b,
               p0wa, p0wb, p0b, p1w, p1b, ow, ob, out_ref):
    agg = parts_ref[0] + parts_ref[1] + xs2_ref[...]
    out2 = d2_ref[...][:, None] * agg + c2b_ref[...]
    h = jnp.maximum(
        jnp.dot(out2, g2w_ref[...], preferred_element_type=jnp.float32,
                precision=_HIGH) + g2b_ref[...], 0.0)
    bf = bf_ref[...]
    oneh = (bf[:, None] == lax.broadcasted_iota(jnp.float32, (1, B), 1)
            ).astype(jnp.float32)
    pooled = lax.dot_general(oneh, h, (((0,), (0,)), ((), ())),
                             preferred_element_type=jnp.float32,
                             precision=_HIGH)
    cnt = jnp.sum(oneh, axis=0)
    h1 = pooled / jnp.maximum(cnt, 1.0)[:, None]

    def dense(v, wref, bref):
        return jnp.maximum(
            jnp.dot(v, wref[...], preferred_element_type=jnp.float32,
                    precision=_HIGH) + bref[...], 0.0)

    h2 = dense(mol_ref[...], m0w, m0b)
    h2 = dense(h2, m1w, m1b)
    h2 = dense(h2, m2w, m2b)
    hh = jnp.maximum(
        jnp.dot(h1, p0wa[...], preferred_element_type=jnp.float32,
                precision=_HIGH)
        + jnp.dot(h2, p0wb[...], preferred_element_type=jnp.float32,
                  precision=_HIGH) + p0b[...], 0.0)
    hh = dense(hh, p1w, p1b)
    out_ref[...] = jnp.dot(hh, ow[...], preferred_element_type=jnp.float32,
                           precision=_HIGH) + ob[...]


def _head(parts2, xs2, d2, c2_bias, g2_w, g2_b, batch_f, mol_features,
          m0_w, m0_b, m1_w, m1_b, m2_w, m2_b,
          p0_wa, p0_wb, p0_b, p1_w, p1_b, o_w, o_b):
    return pl.pallas_call(
        _head_body,
        out_shape=jax.ShapeDtypeStruct((B, 1), jnp.float32),
    )(parts2, xs2, d2, c2_bias, g2_w, g2_b, batch_f, mol_features,
      m0_w, m0_b, m1_w, m1_b, m2_w, m2_b,
      p0_wa, p0_wb, p0_b, p1_w, p1_b, o_w, o_b)


def kernel(x, edge_index, edge_attr, batch_index, mol_features,
           c1_lin_w, c1_mw1, c1_mb1, c1_mw2, c1_mb2, c1_bias,
           g1_w, g1_b,
           c2_lin_w, c2_mw1, c2_mb1, c2_mw2, c2_mb2, c2_bias,
           g2_w, g2_b,
           m0_w, m0_b, m1_w, m1_b, m2_w, m2_b,
           p0_w, p0_b, p1_w, p1_b, o_w, o_b):
    row = edge_index[0]
    col = edge_index[1]

    w1, w2 = _edge_weights(edge_attr, c1_mw1, c1_mb1, c1_mw2, c1_mb2,
                           c2_mw1, c2_mb1, c2_mw2, c2_mb2)
    degp = _degree_partials(col, w1, w2)
    xs1, d1, d2 = _prep1(x, c1_lin_w, degp)
    parts1 = _aggregate(xs1, row, col, w1)
    xs2 = _mid(parts1, xs1, d1, d2, c1_bias, g1_w, g1_b, c2_lin_w)
    parts2 = _aggregate(xs2, row, col, w2)
    return _head(parts2, xs2, d2, c2_bias, g2_w, g2_b,
                 batch_index.astype(jnp.float32), mol_features,
                 m0_w, m0_b, m1_w, m1_b, m2_w, m2_b,
                 p0_w[:D], p0_w[D:], p0_b, p1_w, p1_b, o_w, o_b)


# speedup vs baseline: 12.8641x; 12.8641x over previous
import dataclasses

import jax
import jax.numpy as jnp
from jax import lax
from jax.experimental import pallas as pl
from jax.experimental.pallas import tpu as pltpu
from jax.experimental.pallas import tpu_sc as plsc

N = 10000
E = 320000
B = 256
D = 128
ED = 16
EH = 32

NC = 2
NS = 16
NW = NC * NS
EPW = E // NW
ROWS_PER_TILE = N // NS

_HIGH = lax.Precision.DEFAULT


def _sc_compiler_params():
    cp = pltpu.CompilerParams()
    if "needs_layout_passes" in pltpu.CompilerParams.__dataclass_fields__:
        cp = dataclasses.replace(cp, needs_layout_passes=False)
    return cp


_EBLK = 12800


def _edge_mlp_body(ea_ref, mw1a, mb1a, mw2a, mb2a, mw1b, mb1b, mw2b, mb2b,
                   w1_ref, w2_ref):
    ea = ea_ref[0]

    def mlp(mw1t, mb1, mw2t, mb2):
        h = jnp.maximum(
            jnp.dot(mw1t[...], ea, preferred_element_type=jnp.float32,
                    precision=_HIGH) + mb1[...][:, None], 0.0)
        z = jnp.dot(mw2t[...], h, preferred_element_type=jnp.float32,
                    precision=_HIGH) + mb2[0]
        return jax.nn.sigmoid(z)

    w1_ref[0] = mlp(mw1a, mb1a, mw2a, mb2a)
    w2_ref[0] = mlp(mw1b, mb1b, mw2b, mb2b)


def _edge_weights(edge_attr, mw1a, mb1a, mw2a, mb2a, mw1b, mb1b, mw2b, mb2b):
    full = lambda *s: pl.BlockSpec(s, lambda i: tuple(0 for _ in s))
    nblk = E // _EBLK
    ea_t = jnp.transpose(edge_attr.reshape(nblk, _EBLK, ED), (0, 2, 1))
    w1, w2 = pl.pallas_call(
        _edge_mlp_body,
        grid=(nblk,),
        in_specs=[
            pl.BlockSpec((1, ED, _EBLK), lambda i: (i, 0, 0)),
            full(EH, ED), full(EH), full(1, EH), full(1),
            full(EH, ED), full(EH), full(1, EH), full(1),
        ],
        out_specs=[
            pl.BlockSpec((1, 1, _EBLK), lambda i: (i, 0, 0)),
            pl.BlockSpec((1, 1, _EBLK), lambda i: (i, 0, 0)),
        ],
        out_shape=[
            jax.ShapeDtypeStruct((nblk, 1, _EBLK), jnp.float32),
            jax.ShapeDtypeStruct((nblk, 1, _EBLK), jnp.float32),
        ],
    )(ea_t,
      mw1a.T, mb1a, mw2a.T, mb2a, mw1b.T, mb1b, mw2b.T, mb2b)
    return w1.reshape(E), w2.reshape(E)


_DCHUNK = 2000


def _degree_partials(col, w1, w2):
    mesh = plsc.VectorSubcoreMesh(core_axis_name="c", subcore_axis_name="s")

    @pl.kernel(
        out_type=jax.ShapeDtypeStruct((NW, 2, N), jnp.float32),
        mesh=mesh,
        compiler_params=_sc_compiler_params(),
        scratch_types=[
            pltpu.VMEM((_DCHUNK,), jnp.int32),
            pltpu.VMEM((_DCHUNK,), jnp.float32),
            pltpu.VMEM((_DCHUNK,), jnp.float32),
            pltpu.VMEM((N,), jnp.float32),
            pltpu.VMEM((N,), jnp.float32),
        ],
    )
    def deg_kernel(col_hbm, w1_hbm, w2_hbm, out_hbm,
                   colbuf, w1buf, w2buf, acc1, acc2):
        cid = lax.axis_index("c")
        sid = lax.axis_index("s")
        wid = sid * NC + cid
        base = wid * EPW
        zero = jnp.zeros((16,), jnp.float32)

        @pl.loop(0, N // 16)
        def _(g):
            acc1[pl.ds(g * 16, 16)] = zero
            acc2[pl.ds(g * 16, 16)] = zero

        @pl.loop(0, EPW // _DCHUNK)
        def _(i):
            off = base + i * _DCHUNK
            pltpu.sync_copy(col_hbm.at[pl.ds(off, _DCHUNK)], colbuf)
            pltpu.sync_copy(w1_hbm.at[pl.ds(off, _DCHUNK)], w1buf)
            pltpu.sync_copy(w2_hbm.at[pl.ds(off, _DCHUNK)], w2buf)

            @pl.loop(0, _DCHUNK // 16)
            def _(g):
                idx = colbuf[pl.ds(g * 16, 16)]
                plsc.addupdate_scatter(acc1, [idx], w1buf[pl.ds(g * 16, 16)])
                plsc.addupdate_scatter(acc2, [idx], w2buf[pl.ds(g * 16, 16)])

        pltpu.sync_copy(acc1, out_hbm.at[wid, 0])
        pltpu.sync_copy(acc2, out_hbm.at[wid, 1])

    return deg_kernel(col, w1, w2)


_K = 80


def _aggregate(xs, row, col, w):
    mesh = plsc.VectorSubcoreMesh(core_axis_name="c", subcore_axis_name="s")

    @pl.kernel(
        out_type=jax.ShapeDtypeStruct((NC, N, D), jnp.float32),
        mesh=mesh,
        compiler_params=_sc_compiler_params(),
        scratch_types=[
            pltpu.VMEM((_K,), jnp.int32),
            pltpu.VMEM((_K,), jnp.int32),
            pltpu.VMEM((_K,), jnp.float32),
            pltpu.VMEM((_K, D), jnp.float32),
            pltpu.VMEM_SHARED((N, D), jnp.float32),
        ],
    )
    def agg_kernel(xs_hbm, row_hbm, col_hbm, w_hbm, out_hbm,
                   rowbuf, colbuf, wbuf, rows, acc):
        cid = lax.axis_index("c")
        sid = lax.axis_index("s")
        wid = sid * NC + cid
        base = wid * EPW
        zero = jnp.zeros((16,), jnp.float32)

        @pl.loop(0, _K)
        def _(r):
            for c2 in range(D // 16):
                rows[r, pl.ds(c2 * 16, 16)] = zero

        rbase = sid * ROWS_PER_TILE

        @pl.loop(0, 7)
        def _(i):
            pltpu.sync_copy(rows, acc.at[pl.ds(rbase + i * _K, _K)])

        pltpu.sync_copy(rows.at[pl.ds(0, 65)], acc.at[pl.ds(rbase + 560, 65)])
        plsc.subcore_barrier()

        @pl.loop(0, EPW // _K)
        def _(i):
            off = base + i * _K
            pltpu.sync_copy(row_hbm.at[pl.ds(off, _K)], rowbuf)
            pltpu.sync_copy(col_hbm.at[pl.ds(off, _K)], colbuf)
            pltpu.sync_copy(w_hbm.at[pl.ds(off, _K)], wbuf)
            pltpu.sync_copy(xs_hbm.at[rowbuf], rows)

            @pl.loop(0, _K)
            def _(j):
                jv = jnp.broadcast_to(j, (16,)).astype(jnp.int32)
                wv = plsc.load_gather(wbuf, [jv])
                for c2 in range(D // 16):
                    sl = (j, pl.ds(c2 * 16, 16))
                    rows[sl] = rows[sl] * wv

            pltpu.sync_copy(rows, acc.at[colbuf], add=True)

        plsc.subcore_barrier()

        @pl.when(sid < 15)
        def _():
            pltpu.sync_copy(acc.at[pl.ds(sid * 624, 624)],
                            out_hbm.at[cid, pl.ds(sid * 624, 624)])

        @pl.when(sid == 15)
        def _():
            pltpu.sync_copy(acc.at[pl.ds(9360, 640)],
                            out_hbm.at[cid, pl.ds(9360, 640)])

    return agg_kernel(xs, row, col, w)


def _prep1_body(x_ref, lw_ref, degp_ref, xs_ref, d1_ref, d2_ref):
    degs = jnp.sum(degp_ref[...], axis=0)
    d1 = lax.rsqrt(degs[0] + 1.0)
    d2 = lax.rsqrt(degs[1] + 1.0)
    d1_ref[...] = d1
    d2_ref[...] = d2
    xl = jnp.dot(x_ref[...], lw_ref[...], preferred_element_type=jnp.float32,
                 precision=_HIGH)
    xs_ref[...] = d1[:, None] * xl


def _prep1(x, lin_w, degp):
    return pl.pallas_call(
        _prep1_body,
        out_shape=[
            jax.ShapeDtypeStruct((N, D), jnp.float32),
            jax.ShapeDtypeStruct((N,), jnp.float32),
            jax.ShapeDtypeStruct((N,), jnp.float32),
        ],
    )(x, lin_w, degp)


def _mid_body(parts_ref, xs1_ref, d1_ref, d2_ref, c1b_ref, g1w_ref, g1b_ref,
              lw2_ref, xs2_ref):
    agg = parts_ref[0] + parts_ref[1] + xs1_ref[...]
    out1 = d1_ref[...][:, None] * agg + c1b_ref[...]
    h = jnp.maximum(
        jnp.dot(out1, g1w_ref[...], preferred_element_type=jnp.float32,
                precision=_HIGH) + g1b_ref[...], 0.0)
    xl2 = jnp.dot(h, lw2_ref[...], preferred_element_type=jnp.float32,
                  precision=_HIGH)
    xs2_ref[...] = d2_ref[...][:, None] * xl2


def _mid(parts1, xs1, d1, d2, c1_bias, g1_w, g1_b, c2_lin_w):
    return pl.pallas_call(
        _mid_body,
        out_shape=jax.ShapeDtypeStruct((N, D), jnp.float32),
    )(parts1, xs1, d1, d2, c1_bias, g1_w, g1_b, c2_lin_w)


def _head_body(parts_ref, xs2_ref, d2_ref, c2b_ref, g2w_ref, g2b_ref,
               bf_ref, mol_ref, m0w, m0b, m1w, m1b, m2w, m2b,
               p0wa, p0wb, p0b, p1w, p1b, ow, ob, out_ref):
    agg = parts_ref[0] + parts_ref[1] + xs2_ref[...]
    out2 = d2_ref[...][:, None] * agg + c2b_ref[...]
    h = jnp.maximum(
        jnp.dot(out2, g2w_ref[...], preferred_element_type=jnp.float32,
                precision=_HIGH) + g2b_ref[...], 0.0)
    bf = bf_ref[...]
    oneh = (bf[:, None] == lax.broadcasted_iota(jnp.int32, (1, B), 1
                                                ).astype(jnp.float32)
            ).astype(jnp.float32)
    pooled = lax.dot_general(oneh, h, (((0,), (0,)), ((), ())),
                             preferred_element_type=jnp.float32,
                             precision=_HIGH)
    cnt = jnp.sum(oneh, axis=0)
    h1 = pooled / jnp.maximum(cnt, 1.0)[:, None]

    def dense(v, wref, bref):
        return jnp.maximum(
            jnp.dot(v, wref[...], preferred_element_type=jnp.float32,
                    precision=_HIGH) + bref[...], 0.0)

    h2 = dense(mol_ref[...], m0w, m0b)
    h2 = dense(h2, m1w, m1b)
    h2 = dense(h2, m2w, m2b)
    hh = jnp.maximum(
        jnp.dot(h1, p0wa[...], preferred_element_type=jnp.float32,
                precision=_HIGH)
        + jnp.dot(h2, p0wb[...], preferred_element_type=jnp.float32,
                  precision=_HIGH) + p0b[...], 0.0)
    hh = dense(hh, p1w, p1b)
    out_ref[...] = jnp.dot(hh, ow[...], preferred_element_type=jnp.float32,
                           precision=_HIGH) + ob[...]


def _head(parts2, xs2, d2, c2_bias, g2_w, g2_b, batch_f, mol_features,
          m0_w, m0_b, m1_w, m1_b, m2_w, m2_b,
          p0_wa, p0_wb, p0_b, p1_w, p1_b, o_w, o_b):
    return pl.pallas_call(
        _head_body,
        out_shape=jax.ShapeDtypeStruct((B, 1), jnp.float32),
    )(parts2, xs2, d2, c2_bias, g2_w, g2_b, batch_f, mol_features,
      m0_w, m0_b, m1_w, m1_b, m2_w, m2_b,
      p0_wa, p0_wb, p0_b, p1_w, p1_b, o_w, o_b)


def kernel(x, edge_index, edge_attr, batch_index, mol_features,
           c1_lin_w, c1_mw1, c1_mb1, c1_mw2, c1_mb2, c1_bias,
           g1_w, g1_b,
           c2_lin_w, c2_mw1, c2_mb1, c2_mw2, c2_mb2, c2_bias,
           g2_w, g2_b,
           m0_w, m0_b, m1_w, m1_b, m2_w, m2_b,
           p0_w, p0_b, p1_w, p1_b, o_w, o_b):
    row = edge_index[0]
    col = edge_index[1]

    w1, w2 = _edge_weights(edge_attr, c1_mw1, c1_mb1, c1_mw2, c1_mb2,
                           c2_mw1, c2_mb1, c2_mw2, c2_mb2)
    degp = _degree_partials(col, w1, w2)
    xs1, d1, d2 = _prep1(x, c1_lin_w, degp)
    parts1 = _aggregate(xs1, row, col, w1)
    xs2 = _mid(parts1, xs1, d1, d2, c1_bias, g1_w, g1_b, c2_lin_w)
    parts2 = _aggregate(xs2, row, col, w2)
    return _head(parts2, xs2, d2, c2_bias, g2_w, g2_b,
                 batch_index.astype(jnp.float32), mol_features,
                 m0_w, m0_b, m1_w, m1_b, m2_w, m2_b,
                 p0_w[:D], p0_w[D:], p0_b, p1_w, p1_b, o_w, o_b)


# speedup vs baseline: 22.5312x; 1.7515x over previous
import dataclasses

import jax
import jax.numpy as jnp
from jax import lax
from jax.experimental import pallas as pl
from jax.experimental.pallas import tpu as pltpu
from jax.experimental.pallas import tpu_sc as plsc

N = 10000
E = 320000
B = 256
D = 128
ED = 16
EH = 32

NC = 2
NS = 16
NW = NC * NS
EPW = E // NW
ROWS_PER_TILE = N // NS

_HIGH = lax.Precision.DEFAULT


def _sc_compiler_params():
    cp = pltpu.CompilerParams()
    if "needs_layout_passes" in pltpu.CompilerParams.__dataclass_fields__:
        cp = dataclasses.replace(cp, needs_layout_passes=False)
    return cp


_EBLK = 12800


def _edge_mlp_body(ea_ref, mw1a, mb1a, mw2a, mb2a, mw1b, mb1b, mw2b, mb2b,
                   w1_ref, w2_ref):
    ea = ea_ref[0]

    def mlp(mw1t, mb1, mw2t, mb2):
        h = jnp.maximum(
            jnp.dot(mw1t[...], ea, preferred_element_type=jnp.float32,
                    precision=_HIGH) + mb1[...][:, None], 0.0)
        z = jnp.dot(mw2t[...], h, preferred_element_type=jnp.float32,
                    precision=_HIGH) + mb2[0]
        return jax.nn.sigmoid(z)

    w1_ref[0] = mlp(mw1a, mb1a, mw2a, mb2a)
    w2_ref[0] = mlp(mw1b, mb1b, mw2b, mb2b)


def _edge_weights(edge_attr, mw1a, mb1a, mw2a, mb2a, mw1b, mb1b, mw2b, mb2b):
    full = lambda *s: pl.BlockSpec(s, lambda i: tuple(0 for _ in s))
    nblk = E // _EBLK
    ea_t = jnp.transpose(edge_attr.reshape(nblk, _EBLK, ED), (0, 2, 1))
    w1, w2 = pl.pallas_call(
        _edge_mlp_body,
        grid=(nblk,),
        in_specs=[
            pl.BlockSpec((1, ED, _EBLK), lambda i: (i, 0, 0)),
            full(EH, ED), full(EH), full(1, EH), full(1),
            full(EH, ED), full(EH), full(1, EH), full(1),
        ],
        out_specs=[
            pl.BlockSpec((1, 1, _EBLK), lambda i: (i, 0, 0)),
            pl.BlockSpec((1, 1, _EBLK), lambda i: (i, 0, 0)),
        ],
        out_shape=[
            jax.ShapeDtypeStruct((nblk, 1, _EBLK), jnp.float32),
            jax.ShapeDtypeStruct((nblk, 1, _EBLK), jnp.float32),
        ],
    )(ea_t,
      mw1a.T, mb1a, mw2a.T, mb2a, mw1b.T, mb1b, mw2b.T, mb2b)
    return w1.reshape(E), w2.reshape(E)


_DCHUNK = 2000


def _degree_partials(col, w1, w2):
    mesh = plsc.VectorSubcoreMesh(core_axis_name="c", subcore_axis_name="s")

    @pl.kernel(
        out_type=jax.ShapeDtypeStruct((NW, 2, N), jnp.float32),
        mesh=mesh,
        compiler_params=_sc_compiler_params(),
        scratch_types=[
            pltpu.VMEM((_DCHUNK,), jnp.int32),
            pltpu.VMEM((_DCHUNK,), jnp.float32),
            pltpu.VMEM((_DCHUNK,), jnp.float32),
            pltpu.VMEM((N,), jnp.float32),
            pltpu.VMEM((N,), jnp.float32),
        ],
    )
    def deg_kernel(col_hbm, w1_hbm, w2_hbm, out_hbm,
                   colbuf, w1buf, w2buf, acc1, acc2):
        cid = lax.axis_index("c")
        sid = lax.axis_index("s")
        wid = sid * NC + cid
        base = wid * EPW
        zero = jnp.zeros((16,), jnp.float32)

        @pl.loop(0, N // 16)
        def _(g):
            acc1[pl.ds(g * 16, 16)] = zero
            acc2[pl.ds(g * 16, 16)] = zero

        @pl.loop(0, EPW // _DCHUNK)
        def _(i):
            off = base + i * _DCHUNK
            pltpu.sync_copy(col_hbm.at[pl.ds(off, _DCHUNK)], colbuf)
            pltpu.sync_copy(w1_hbm.at[pl.ds(off, _DCHUNK)], w1buf)
            pltpu.sync_copy(w2_hbm.at[pl.ds(off, _DCHUNK)], w2buf)

            @pl.loop(0, _DCHUNK // 16)
            def _(g):
                idx = colbuf[pl.ds(g * 16, 16)]
                plsc.addupdate_scatter(acc1, [idx], w1buf[pl.ds(g * 16, 16)])
                plsc.addupdate_scatter(acc2, [idx], w2buf[pl.ds(g * 16, 16)])

        pltpu.sync_copy(acc1, out_hbm.at[wid, 0])
        pltpu.sync_copy(acc2, out_hbm.at[wid, 1])

    return deg_kernel(col, w1, w2)


_K = 40
_NIT = EPW // _K


def _aggregate(xs, row, col, w):
    mesh = plsc.VectorSubcoreMesh(core_axis_name="c", subcore_axis_name="s")

    @pl.kernel(
        out_type=jax.ShapeDtypeStruct((NC, N, D), jnp.float32),
        mesh=mesh,
        compiler_params=_sc_compiler_params(),
        scratch_types=[
            pltpu.VMEM((_K,), jnp.int32),
            pltpu.VMEM((_K,), jnp.int32),
            pltpu.VMEM((_K,), jnp.int32),
            pltpu.VMEM((_K,), jnp.int32),
            pltpu.VMEM((_K,), jnp.int32),
            pltpu.VMEM((_K,), jnp.int32),
            pltpu.VMEM((_K,), jnp.float32),
            pltpu.VMEM((_K,), jnp.float32),
            pltpu.VMEM((_K, D), jnp.float32),
            pltpu.VMEM((_K, D), jnp.float32),
            pltpu.VMEM_SHARED((N, D), jnp.float32),
            pltpu.SemaphoreType.DMA,
            pltpu.SemaphoreType.DMA,
            pltpu.SemaphoreType.DMA,
            pltpu.SemaphoreType.DMA,
            pltpu.SemaphoreType.DMA,
            pltpu.SemaphoreType.DMA,
        ],
    )
    def agg_kernel(xs_hbm, row_hbm, col_hbm, w_hbm, out_hbm,
                   rowb0, rowb1, colb0, colb1, colsc0, colsc1,
                   wb0, wb1, rows0, rows1, acc,
                   isem0, isem1, gsem0, gsem1, ssem0, ssem1):
        rowbufs = (rowb0, rowb1)
        colbufs = (colb0, colb1)
        colscs = (colsc0, colsc1)
        wbufs = (wb0, wb1)
        rowsb = (rows0, rows1)
        isems = (isem0, isem1)
        gsems = (gsem0, gsem1)
        ssems = (ssem0, ssem1)

        cid = lax.axis_index("c")
        sid = lax.axis_index("s")
        wid = sid * NC + cid
        base = wid * EPW
        zero = jnp.zeros((16,), jnp.float32)

        def issue_idx(ci, b):
            off = base + ci * _K
            pltpu.async_copy(row_hbm.at[pl.ds(off, _K)], rowbufs[b], isems[b])
            pltpu.async_copy(col_hbm.at[pl.ds(off, _K)], colbufs[b], isems[b])
            pltpu.async_copy(w_hbm.at[pl.ds(off, _K)], wbufs[b], isems[b])

        def wait_idx(ci, b):
            off = base + ci * _K
            pltpu.make_async_copy(row_hbm.at[pl.ds(off, _K)], rowbufs[b],
                                  isems[b]).wait()
            pltpu.make_async_copy(col_hbm.at[pl.ds(off, _K)], colbufs[b],
                                  isems[b]).wait()
            pltpu.make_async_copy(w_hbm.at[pl.ds(off, _K)], wbufs[b],
                                  isems[b]).wait()

        def issue_gather(b):
            pltpu.async_copy(xs_hbm.at[rowbufs[b]], rowsb[b], gsems[b])

        def wait_gather(b):
            pltpu.make_async_copy(xs_hbm.at[rowbufs[b]], rowsb[b],
                                  gsems[b]).wait()

        def issue_scatter(b):
            pltpu.async_copy(rowsb[b], acc.at[colscs[b]], ssems[b], add=True)

        def wait_scatter(b):
            pltpu.make_async_copy(rowsb[b], acc.at[colscs[b]],
                                  ssems[b]).wait()

        def scale(b):
            rb = rowsb[b]
            wb = wbufs[b]

            @pl.loop(0, _K // 16)
            def _(g):
                g16 = g * 16
                wv = wb[pl.ds(g16, 16)]
                for j in range(16):
                    wj = wv.at[jnp.full((16,), j, jnp.int32)].get(
                        mode="promise_in_bounds")
                    r = g16 + j
                    for c2 in range(D // 16):
                        sl = (r, pl.ds(c2 * 16, 16))
                        rb[sl] = rb[sl] * wj

        def snapshot_col(b):
            for g in range(_K // 16):
                colscs[b][pl.ds(g * 16, 16)] = colbufs[b][pl.ds(g * 16, 16)]

        @pl.loop(0, _K)
        def _(r):
            for c2 in range(D // 16):
                rows0[r, pl.ds(c2 * 16, 16)] = zero

        rbase = sid * ROWS_PER_TILE

        @pl.loop(0, 15)
        def _(i):
            pltpu.sync_copy(rows0, acc.at[pl.ds(rbase + i * _K, _K)])

        pltpu.sync_copy(rows0.at[pl.ds(0, 25)], acc.at[pl.ds(rbase + 600, 25)])
        plsc.subcore_barrier()

        issue_idx(0, 0)
        issue_idx(1, 1)
        wait_idx(0, 0)
        issue_gather(0)

        @pl.loop(0, _NIT, step=2)
        def _(i):
            for b in range(2):
                ci = i + b
                b1 = 1 - b
                wait_gather(b)

                @pl.when(ci + 1 < _NIT)
                def _():
                    wait_idx(ci + 1, b1)

                    @pl.when(ci >= 1)
                    def _():
                        wait_scatter(b1)

                    issue_gather(b1)

                scale(b)
                snapshot_col(b)
                issue_scatter(b)

                @pl.when(ci + 2 < _NIT)
                def _():
                    issue_idx(ci + 2, b)

        wait_scatter(0)
        wait_scatter(1)
        plsc.subcore_barrier()

        @pl.when(sid < 15)
        def _():
            pltpu.sync_copy(acc.at[pl.ds(sid * 624, 624)],
                            out_hbm.at[cid, pl.ds(sid * 624, 624)])

        @pl.when(sid == 15)
        def _():
            pltpu.sync_copy(acc.at[pl.ds(9360, 640)],
                            out_hbm.at[cid, pl.ds(9360, 640)])

    return agg_kernel(xs, row, col, w)


def _prep1_body(x_ref, lw_ref, degp_ref, xs_ref, d1_ref, d2_ref):
    degs = jnp.sum(degp_ref[...], axis=0)
    d1 = lax.rsqrt(degs[0] + 1.0)
    d2 = lax.rsqrt(degs[1] + 1.0)
    d1_ref[...] = d1
    d2_ref[...] = d2
    xl = jnp.dot(x_ref[...], lw_ref[...], preferred_element_type=jnp.float32,
                 precision=_HIGH)
    xs_ref[...] = d1[:, None] * xl


def _prep1(x, lin_w, degp):
    return pl.pallas_call(
        _prep1_body,
        out_shape=[
            jax.ShapeDtypeStruct((N, D), jnp.float32),
            jax.ShapeDtypeStruct((N,), jnp.float32),
            jax.ShapeDtypeStruct((N,), jnp.float32),
        ],
    )(x, lin_w, degp)


def _mid_body(parts_ref, xs1_ref, d1_ref, d2_ref, c1b_ref, g1w_ref, g1b_ref,
              lw2_ref, xs2_ref):
    agg = parts_ref[0] + parts_ref[1] + xs1_ref[...]
    out1 = d1_ref[...][:, None] * agg + c1b_ref[...]
    h = jnp.maximum(
        jnp.dot(out1, g1w_ref[...], preferred_element_type=jnp.float32,
                precision=_HIGH) + g1b_ref[...], 0.0)
    xl2 = jnp.dot(h, lw2_ref[...], preferred_element_type=jnp.float32,
                  precision=_HIGH)
    xs2_ref[...] = d2_ref[...][:, None] * xl2


def _mid(parts1, xs1, d1, d2, c1_bias, g1_w, g1_b, c2_lin_w):
    return pl.pallas_call(
        _mid_body,
        out_shape=jax.ShapeDtypeStruct((N, D), jnp.float32),
    )(parts1, xs1, d1, d2, c1_bias, g1_w, g1_b, c2_lin_w)


def _head_body(parts_ref, xs2_ref, d2_ref, c2b_ref, g2w_ref, g2b_ref,
               bf_ref, mol_ref, m0w, m0b, m1w, m1b, m2w, m2b,
               p0wa, p0wb, p0b, p1w, p1b, ow, ob, out_ref):
    agg = parts_ref[0] + parts_ref[1] + xs2_ref[...]
    out2 = d2_ref[...][:, None] * agg + c2b_ref[...]
    h = jnp.maximum(
        jnp.dot(out2, g2w_ref[...], preferred_element_type=jnp.float32,
                precision=_HIGH) + g2b_ref[...], 0.0)
    bf = bf_ref[...]
    oneh = (bf[:, None] == lax.broadcasted_iota(jnp.int32, (1, B), 1
                                                ).astype(jnp.float32)
            ).astype(jnp.float32)
    pooled = lax.dot_general(oneh, h, (((0,), (0,)), ((), ())),
                             preferred_element_type=jnp.float32,
                             precision=_HIGH)
    cnt = jnp.sum(oneh, axis=0)
    h1 = pooled / jnp.maximum(cnt, 1.0)[:, None]

    def dense(v, wref, bref):
        return jnp.maximum(
            jnp.dot(v, wref[...], preferred_element_type=jnp.float32,
                    precision=_HIGH) + bref[...], 0.0)

    h2 = dense(mol_ref[...], m0w, m0b)
    h2 = dense(h2, m1w, m1b)
    h2 = dense(h2, m2w, m2b)
    hh = jnp.maximum(
        jnp.dot(h1, p0wa[...], preferred_element_type=jnp.float32,
                precision=_HIGH)
        + jnp.dot(h2, p0wb[...], preferred_element_type=jnp.float32,
                  precision=_HIGH) + p0b[...], 0.0)
    hh = dense(hh, p1w, p1b)
    out_ref[...] = jnp.dot(hh, ow[...], preferred_element_type=jnp.float32,
                           precision=_HIGH) + ob[...]


def _head(parts2, xs2, d2, c2_bias, g2_w, g2_b, batch_f, mol_features,
          m0_w, m0_b, m1_w, m1_b, m2_w, m2_b,
          p0_wa, p0_wb, p0_b, p1_w, p1_b, o_w, o_b):
    return pl.pallas_call(
        _head_body,
        out_shape=jax.ShapeDtypeStruct((B, 1), jnp.float32),
    )(parts2, xs2, d2, c2_bias, g2_w, g2_b, batch_f, mol_features,
      m0_w, m0_b, m1_w, m1_b, m2_w, m2_b,
      p0_wa, p0_wb, p0_b, p1_w, p1_b, o_w, o_b)


def kernel(x, edge_index, edge_attr, batch_index, mol_features,
           c1_lin_w, c1_mw1, c1_mb1, c1_mw2, c1_mb2, c1_bias,
           g1_w, g1_b,
           c2_lin_w, c2_mw1, c2_mb1, c2_mw2, c2_mb2, c2_bias,
           g2_w, g2_b,
           m0_w, m0_b, m1_w, m1_b, m2_w, m2_b,
           p0_w, p0_b, p1_w, p1_b, o_w, o_b):
    row = edge_index[0]
    col = edge_index[1]

    w1, w2 = _edge_weights(edge_attr, c1_mw1, c1_mb1, c1_mw2, c1_mb2,
                           c2_mw1, c2_mb1, c2_mw2, c2_mb2)
    degp = _degree_partials(col, w1, w2)
    xs1, d1, d2 = _prep1(x, c1_lin_w, degp)
    parts1 = _aggregate(xs1, row, col, w1)
    xs2 = _mid(parts1, xs1, d1, d2, c1_bias, g1_w, g1_b, c2_lin_w)
    parts2 = _aggregate(xs2, row, col, w2)
    return _head(parts2, xs2, d2, c2_bias, g2_w, g2_b,
                 batch_index.astype(jnp.float32), mol_features,
                 m0_w, m0_b, m1_w, m1_b, m2_w, m2_b,
                 p0_w[:D], p0_w[D:], p0_b, p1_w, p1_b, o_w, o_b)


# speedup vs baseline: 32.4186x; 1.4388x over previous
import dataclasses

import jax
import jax.numpy as jnp
from jax import lax
from jax.experimental import pallas as pl
from jax.experimental.pallas import tpu as pltpu
from jax.experimental.pallas import tpu_sc as plsc

N = 10000
E = 320000
B = 256
D = 128
ED = 16
EH = 32

NC = 2
NS = 16
NW = NC * NS
EPW = E // NW
ROWS_PER_TILE = N // NS

_HIGH = lax.Precision.DEFAULT


def _sc_compiler_params():
    cp = pltpu.CompilerParams()
    if "needs_layout_passes" in pltpu.CompilerParams.__dataclass_fields__:
        cp = dataclasses.replace(cp, needs_layout_passes=False)
    return cp


_EBLK = 12800


def _edge_mlp_body(ea_ref, mw1a, mb1a, mw2a, mb2a, mw1b, mb1b, mw2b, mb2b,
                   w1_ref, w2_ref):
    ea = ea_ref[0]

    def mlp(mw1t, mb1, mw2t, mb2):
        h = jnp.maximum(
            jnp.dot(mw1t[...], ea, preferred_element_type=jnp.float32,
                    precision=_HIGH) + mb1[...][:, None], 0.0)
        z = jnp.dot(mw2t[...], h, preferred_element_type=jnp.float32,
                    precision=_HIGH) + mb2[0]
        return jax.nn.sigmoid(z)

    w1_ref[0] = mlp(mw1a, mb1a, mw2a, mb2a)
    w2_ref[0] = mlp(mw1b, mb1b, mw2b, mb2b)


def _edge_weights(edge_attr, mw1a, mb1a, mw2a, mb2a, mw1b, mb1b, mw2b, mb2b):
    full = lambda *s: pl.BlockSpec(s, lambda i: tuple(0 for _ in s))
    nblk = E // _EBLK
    ea_t = jnp.transpose(edge_attr.reshape(nblk, _EBLK, ED), (0, 2, 1))
    w1, w2 = pl.pallas_call(
        _edge_mlp_body,
        grid=(nblk,),
        in_specs=[
            pl.BlockSpec((1, ED, _EBLK), lambda i: (i, 0, 0)),
            full(EH, ED), full(EH), full(1, EH), full(1),
            full(EH, ED), full(EH), full(1, EH), full(1),
        ],
        out_specs=[
            pl.BlockSpec((1, 1, _EBLK), lambda i: (i, 0, 0)),
            pl.BlockSpec((1, 1, _EBLK), lambda i: (i, 0, 0)),
        ],
        out_shape=[
            jax.ShapeDtypeStruct((nblk, 1, _EBLK), jnp.float32),
            jax.ShapeDtypeStruct((nblk, 1, _EBLK), jnp.float32),
        ],
    )(ea_t,
      mw1a.T, mb1a, mw2a.T, mb2a, mw1b.T, mb1b, mw2b.T, mb2b)
    return w1.reshape(E), w2.reshape(E)


_DCHUNK = 2000


def _degree_partials(col, w1, w2):
    mesh = plsc.VectorSubcoreMesh(core_axis_name="c", subcore_axis_name="s")

    @pl.kernel(
        out_type=jax.ShapeDtypeStruct((NW, 2, N), jnp.float32),
        mesh=mesh,
        compiler_params=_sc_compiler_params(),
        scratch_types=[
            pltpu.VMEM((_DCHUNK,), jnp.int32),
            pltpu.VMEM((_DCHUNK,), jnp.float32),
            pltpu.VMEM((_DCHUNK,), jnp.float32),
            pltpu.VMEM((N,), jnp.float32),
            pltpu.VMEM((N,), jnp.float32),
        ],
    )
    def deg_kernel(col_hbm, w1_hbm, w2_hbm, out_hbm,
                   colbuf, w1buf, w2buf, acc1, acc2):
        cid = lax.axis_index("c")
        sid = lax.axis_index("s")
        wid = sid * NC + cid
        base = wid * EPW
        zero = jnp.zeros((16,), jnp.float32)

        @pl.loop(0, N // 16)
        def _(g):
            acc1[pl.ds(g * 16, 16)] = zero
            acc2[pl.ds(g * 16, 16)] = zero

        @pl.loop(0, EPW // _DCHUNK)
        def _(i):
            off = base + i * _DCHUNK
            pltpu.sync_copy(col_hbm.at[pl.ds(off, _DCHUNK)], colbuf)
            pltpu.sync_copy(w1_hbm.at[pl.ds(off, _DCHUNK)], w1buf)
            pltpu.sync_copy(w2_hbm.at[pl.ds(off, _DCHUNK)], w2buf)

            @pl.loop(0, _DCHUNK // 16)
            def _(g):
                idx = colbuf[pl.ds(g * 16, 16)]
                plsc.addupdate_scatter(acc1, [idx], w1buf[pl.ds(g * 16, 16)])
                plsc.addupdate_scatter(acc2, [idx], w2buf[pl.ds(g * 16, 16)])

        pltpu.sync_copy(acc1, out_hbm.at[wid, 0])
        pltpu.sync_copy(acc2, out_hbm.at[wid, 1])

    return deg_kernel(col, w1, w2)


_K = 80
_NIT = EPW // _K
_NBUF = 4


def _aggregate(xs, row, col, w):
    mesh = plsc.VectorSubcoreMesh(core_axis_name="c", subcore_axis_name="s")

    @pl.kernel(
        out_type=jax.ShapeDtypeStruct((NC, N, D), jnp.float32),
        mesh=mesh,
        compiler_params=_sc_compiler_params(),
        scratch_types=(
            [pltpu.VMEM((_K,), jnp.int32)] * _NBUF
            + [pltpu.VMEM((_K,), jnp.int32)] * _NBUF
            + [pltpu.VMEM((_K,), jnp.float32)] * _NBUF
            + [pltpu.VMEM((_K, D), jnp.float32)] * _NBUF
            + [pltpu.VMEM_SHARED((N, D), jnp.float32)]
            + [pltpu.SemaphoreType.DMA] * (3 * _NBUF)
        ),
    )
    def agg_kernel(xs_hbm, row_hbm, col_hbm, w_hbm, out_hbm, *scr):
        rowbufs = scr[0:_NBUF]
        colbufs = scr[_NBUF:2 * _NBUF]
        wbufs = scr[2 * _NBUF:3 * _NBUF]
        rowsb = scr[3 * _NBUF:4 * _NBUF]
        acc = scr[4 * _NBUF]
        isems = scr[4 * _NBUF + 1:5 * _NBUF + 1]
        gsems = scr[5 * _NBUF + 1:6 * _NBUF + 1]
        ssems = scr[6 * _NBUF + 1:7 * _NBUF + 1]

        cid = lax.axis_index("c")
        sid = lax.axis_index("s")
        wid = sid * NC + cid
        base = wid * EPW
        zero = jnp.zeros((16,), jnp.float32)

        def issue_idx(ci, b):
            off = base + ci * _K
            pltpu.async_copy(row_hbm.at[pl.ds(off, _K)], rowbufs[b], isems[b])
            pltpu.async_copy(col_hbm.at[pl.ds(off, _K)], colbufs[b], isems[b])
            pltpu.async_copy(w_hbm.at[pl.ds(off, _K)], wbufs[b], isems[b])

        def wait_idx(ci, b):
            off = base + ci * _K
            pltpu.make_async_copy(row_hbm.at[pl.ds(off, _K)], rowbufs[b],
                                  isems[b]).wait()
            pltpu.make_async_copy(col_hbm.at[pl.ds(off, _K)], colbufs[b],
                                  isems[b]).wait()
            pltpu.make_async_copy(w_hbm.at[pl.ds(off, _K)], wbufs[b],
                                  isems[b]).wait()

        def issue_gather(b):
            pltpu.async_copy(xs_hbm.at[rowbufs[b]], rowsb[b], gsems[b])

        def wait_gather(b):
            pltpu.make_async_copy(xs_hbm.at[rowbufs[b]], rowsb[b],
                                  gsems[b]).wait()

        def issue_scatter(b):
            pltpu.async_copy(rowsb[b], acc.at[colbufs[b]], ssems[b], add=True)

        def wait_scatter(b):
            pltpu.make_async_copy(rowsb[b], acc.at[colbufs[b]],
                                  ssems[b]).wait()

        def scale(b):
            rb = rowsb[b]
            wb = wbufs[b]

            @pl.loop(0, _K // 16)
            def _(g):
                g16 = g * 16
                wv = wb[pl.ds(g16, 16)]
                for j in range(16):
                    wj = wv.at[jnp.full((16,), j, jnp.int32)].get(
                        mode="promise_in_bounds")
                    r = g16 + j
                    for c2 in range(D // 16):
                        sl = (r, pl.ds(c2 * 16, 16))
                        rb[sl] = rb[sl] * wj

        @pl.loop(0, _K)
        def _(r):
            for c2 in range(D // 16):
                rowsb[0][r, pl.ds(c2 * 16, 16)] = zero

        rbase = sid * ROWS_PER_TILE

        @pl.loop(0, 7)
        def _(i):
            pltpu.sync_copy(rowsb[0], acc.at[pl.ds(rbase + i * _K, _K)])

        pltpu.sync_copy(rowsb[0].at[pl.ds(0, 65)],
                        acc.at[pl.ds(rbase + 560, 65)])
        plsc.subcore_barrier()

        def body(ci, b):
            wait_gather(b)

            @pl.when(ci + 2 < _NIT)
            def _():
                wait_idx(ci + 2, (b + 2) % _NBUF)
                issue_gather((b + 2) % _NBUF)

            scale(b)
            issue_scatter(b)

            @pl.when(ci >= 1)
            def _():
                wait_scatter((b + 3) % _NBUF)

            @pl.when(ci + 3 < _NIT)
            def _():
                issue_idx(ci + 3, (b + 3) % _NBUF)

        issue_idx(0, 0)
        issue_idx(1, 1)
        issue_idx(2, 2)
        wait_idx(0, 0)
        issue_gather(0)
        wait_idx(1, 1)
        issue_gather(1)

        @pl.loop(0, _NIT - 1, step=_NBUF)
        def _(i):
            for b in range(_NBUF):
                body(i + b, b)

        wait_gather(0)
        scale(0)
        issue_scatter(0)
        wait_scatter(3)
        wait_scatter(0)
        plsc.subcore_barrier()

        @pl.when(sid < 15)
        def _():
            pltpu.sync_copy(acc.at[pl.ds(sid * 624, 624)],
                            out_hbm.at[cid, pl.ds(sid * 624, 624)])

        @pl.when(sid == 15)
        def _():
            pltpu.sync_copy(acc.at[pl.ds(9360, 640)],
                            out_hbm.at[cid, pl.ds(9360, 640)])

    return agg_kernel(xs, row, col, w)


def _prep1_body(x_ref, lw_ref, degp_ref, xs_ref, d1_ref, d2_ref):
    degs = jnp.sum(degp_ref[...], axis=0)
    d1 = lax.rsqrt(degs[0] + 1.0)
    d2 = lax.rsqrt(degs[1] + 1.0)
    d1_ref[...] = d1
    d2_ref[...] = d2
    xl = jnp.dot(x_ref[...], lw_ref[...], preferred_element_type=jnp.float32,
                 precision=_HIGH)
    xs_ref[...] = d1[:, None] * xl


def _prep1(x, lin_w, degp):
    return pl.pallas_call(
        _prep1_body,
        out_shape=[
            jax.ShapeDtypeStruct((N, D), jnp.float32),
            jax.ShapeDtypeStruct((N,), jnp.float32),
            jax.ShapeDtypeStruct((N,), jnp.float32),
        ],
    )(x, lin_w, degp)


def _mid_body(parts_ref, xs1_ref, d1_ref, d2_ref, c1b_ref, g1w_ref, g1b_ref,
              lw2_ref, xs2_ref):
    agg = parts_ref[0] + parts_ref[1] + xs1_ref[...]
    out1 = d1_ref[...][:, None] * agg + c1b_ref[...]
    h = jnp.maximum(
        jnp.dot(out1, g1w_ref[...], preferred_element_type=jnp.float32,
                precision=_HIGH) + g1b_ref[...], 0.0)
    xl2 = jnp.dot(h, lw2_ref[...], preferred_element_type=jnp.float32,
                  precision=_HIGH)
    xs2_ref[...] = d2_ref[...][:, None] * xl2


def _mid(parts1, xs1, d1, d2, c1_bias, g1_w, g1_b, c2_lin_w):
    return pl.pallas_call(
        _mid_body,
        out_shape=jax.ShapeDtypeStruct((N, D), jnp.float32),
    )(parts1, xs1, d1, d2, c1_bias, g1_w, g1_b, c2_lin_w)


def _head_body(parts_ref, xs2_ref, d2_ref, c2b_ref, g2w_ref, g2b_ref,
               bf_ref, mol_ref, m0w, m0b, m1w, m1b, m2w, m2b,
               p0wa, p0wb, p0b, p1w, p1b, ow, ob, out_ref):
    agg = parts_ref[0] + parts_ref[1] + xs2_ref[...]
    out2 = d2_ref[...][:, None] * agg + c2b_ref[...]
    h = jnp.maximum(
        jnp.dot(out2, g2w_ref[...], preferred_element_type=jnp.float32,
                precision=_HIGH) + g2b_ref[...], 0.0)
    bf = bf_ref[...]
    oneh = (bf[:, None] == lax.broadcasted_iota(jnp.int32, (1, B), 1
                                                ).astype(jnp.float32)
            ).astype(jnp.float32)
    pooled = lax.dot_general(oneh, h, (((0,), (0,)), ((), ())),
                             preferred_element_type=jnp.float32,
                             precision=_HIGH)
    cnt = jnp.sum(oneh, axis=0)
    h1 = pooled / jnp.maximum(cnt, 1.0)[:, None]

    def dense(v, wref, bref):
        return jnp.maximum(
            jnp.dot(v, wref[...], preferred_element_type=jnp.float32,
                    precision=_HIGH) + bref[...], 0.0)

    h2 = dense(mol_ref[...], m0w, m0b)
    h2 = dense(h2, m1w, m1b)
    h2 = dense(h2, m2w, m2b)
    hh = jnp.maximum(
        jnp.dot(h1, p0wa[...], preferred_element_type=jnp.float32,
                precision=_HIGH)
        + jnp.dot(h2, p0wb[...], preferred_element_type=jnp.float32,
                  precision=_HIGH) + p0b[...], 0.0)
    hh = dense(hh, p1w, p1b)
    out_ref[...] = jnp.dot(hh, ow[...], preferred_element_type=jnp.float32,
                           precision=_HIGH) + ob[...]


def _head(parts2, xs2, d2, c2_bias, g2_w, g2_b, batch_f, mol_features,
          m0_w, m0_b, m1_w, m1_b, m2_w, m2_b,
          p0_wa, p0_wb, p0_b, p1_w, p1_b, o_w, o_b):
    return pl.pallas_call(
        _head_body,
        out_shape=jax.ShapeDtypeStruct((B, 1), jnp.float32),
    )(parts2, xs2, d2, c2_bias, g2_w, g2_b, batch_f, mol_features,
      m0_w, m0_b, m1_w, m1_b, m2_w, m2_b,
      p0_wa, p0_wb, p0_b, p1_w, p1_b, o_w, o_b)


def kernel(x, edge_index, edge_attr, batch_index, mol_features,
           c1_lin_w, c1_mw1, c1_mb1, c1_mw2, c1_mb2, c1_bias,
           g1_w, g1_b,
           c2_lin_w, c2_mw1, c2_mb1, c2_mw2, c2_mb2, c2_bias,
           g2_w, g2_b,
           m0_w, m0_b, m1_w, m1_b, m2_w, m2_b,
           p0_w, p0_b, p1_w, p1_b, o_w, o_b):
    row = edge_index[0]
    col = edge_index[1]

    w1, w2 = _edge_weights(edge_attr, c1_mw1, c1_mb1, c1_mw2, c1_mb2,
                           c2_mw1, c2_mb1, c2_mw2, c2_mb2)
    degp = _degree_partials(col, w1, w2)
    xs1, d1, d2 = _prep1(x, c1_lin_w, degp)
    parts1 = _aggregate(xs1, row, col, w1)
    xs2 = _mid(parts1, xs1, d1, d2, c1_bias, g1_w, g1_b, c2_lin_w)
    parts2 = _aggregate(xs2, row, col, w2)
    return _head(parts2, xs2, d2, c2_bias, g2_w, g2_b,
                 batch_index.astype(jnp.float32), mol_features,
                 m0_w, m0_b, m1_w, m1_b, m2_w, m2_b,
                 p0_w[:D], p0_w[D:], p0_b, p1_w, p1_b, o_w, o_b)
